# Initial kernel scaffold; baseline (speedup 1.0000x reference)
import jax
import jax.numpy as jnp
from jax.experimental import pallas as pl


def kernel(node_features, batch_vector, W1, b1, W2, b2, Wout, bout):
    raise NotImplementedError("write your pallas kernel here")


# speedup vs baseline: 3.7239x; 3.7239x over previous
import jax
import jax.numpy as jnp
from jax.experimental import pallas as pl
from jax.experimental.pallas import tpu as pltpu

_N, _D, _H, _O, _B = 100000, 128, 128, 128, 1024
_BN = 2000
_NB = _N // _BN


def _fused_body(ids_ref, x_ref, w1_ref, b1_ref, w2_ref, b2_ref,
                wout_ref, bout_ref, out_ref, acc_ref):
    g = pl.program_id(0)

    @pl.when(g == 0)
    def _init():
        acc_ref[...] = jnp.zeros_like(acc_ref)

    x = x_ref[...]
    h = jnp.dot(x, w1_ref[...], preferred_element_type=jnp.float32)
    h = jnp.maximum(h + b1_ref[...], 0.0)
    h = jnp.dot(h, w2_ref[...], preferred_element_type=jnp.float32)
    h = jnp.maximum(h + b2_ref[...], 0.0)

    ids = ids_ref[0, 0, :]
    onehot = (jax.lax.broadcasted_iota(jnp.int32, (_BN, _B), 1)
              == ids[:, None]).astype(jnp.float32)
    partial = jax.lax.dot_general(
        onehot, h, (((0,), (0,)), ((), ())),
        preferred_element_type=jnp.float32)
    acc_ref[...] += partial

    @pl.when(g == _NB - 1)
    def _final():
        out_ref[...] = (
            jnp.dot(acc_ref[...], wout_ref[...],
                    preferred_element_type=jnp.float32) + bout_ref[...])


def kernel(node_features, batch_vector, W1, b1, W2, b2, Wout, bout):
    ids3 = batch_vector.astype(jnp.int32).reshape(_NB, 1, _BN)
    return pl.pallas_call(
        _fused_body,
        grid=(_NB,),
        in_specs=[
            pl.BlockSpec((1, 1, _BN), lambda g: (g, 0, 0)),
            pl.BlockSpec((_BN, _D), lambda g: (g, 0)),
            pl.BlockSpec((_D, _H), lambda g: (0, 0)),
            pl.BlockSpec((1, _H), lambda g: (0, 0)),
            pl.BlockSpec((_H, _H), lambda g: (0, 0)),
            pl.BlockSpec((1, _H), lambda g: (0, 0)),
            pl.BlockSpec((_H, _O), lambda g: (0, 0)),
            pl.BlockSpec((1, _O), lambda g: (0, 0)),
        ],
        out_specs=pl.BlockSpec((_B, _O), lambda g: (0, 0)),
        out_shape=jax.ShapeDtypeStruct((_B, _O), jnp.float32),
        scratch_shapes=[pltpu.VMEM((_B, _O), jnp.float32)],
    )(ids3, node_features, W1, b1.reshape(1, _H), W2, b2.reshape(1, _H),
      Wout, bout.reshape(1, _O))
